# Initial kernel scaffold; baseline (speedup 1.0000x reference)
import jax
import jax.numpy as jnp
from jax.experimental import pallas as pl


def kernel(x, adj, W, attn_src, attn_dst):
    raise NotImplementedError("write your pallas kernel here")


# speedup vs baseline: 1.9553x; 1.9553x over previous
import jax
import jax.numpy as jnp
from jax.experimental import pallas as pl
from jax.experimental.pallas import tpu as pltpu

N = 8192
D = 256
ALPHA = 0.2
RB_H = 512
RB = 128


def _h_kernel(x_ref, w_ref, asrc_ref, adst_ref, hbf_ref, av_src_ref, av_dst_ref):
    h = jax.lax.dot_general(
        x_ref[...], w_ref[...],
        dimension_numbers=(((1,), (1,)), ((), ())),
        preferred_element_type=jnp.float32,
    )
    hbf_ref[...] = h.astype(jnp.bfloat16)
    av_src_ref[...] = jnp.sum(h * asrc_ref[...], axis=1, keepdims=True)
    av_dst_ref[...] = jnp.sum(h * adst_ref[...], axis=1, keepdims=True)


def _attn_kernel(adj_ref, asrc_row_ref, adst_col_ref, hbf_ref, out_ref):
    s = asrc_row_ref[...] + adst_col_ref[...]
    s = jnp.where(s > 0, s, ALPHA * s)
    mask = adj_ref[...] > 0
    s = jnp.where(mask, s, 0.0)
    rowmax = jnp.max(s, axis=1, keepdims=True)
    e = jnp.where(mask, jnp.exp(s - rowmax), 0.0)
    denom = jnp.sum(e, axis=1, keepdims=True) + 1e-8
    w = (e / denom).astype(jnp.bfloat16)
    out = jnp.dot(w, hbf_ref[...], preferred_element_type=jnp.float32)
    out_ref[...] = jnp.where(out > 0, out, jnp.exp(jnp.minimum(out, 0.0)) - 1.0)


def kernel(x, adj, W, attn_src, attn_dst):
    hbf, a_src, a_dst = pl.pallas_call(
        _h_kernel,
        grid=(N // RB_H,),
        in_specs=[
            pl.BlockSpec((RB_H, D), lambda i: (i, 0)),
            pl.BlockSpec((D, D), lambda i: (0, 0)),
            pl.BlockSpec((1, D), lambda i: (0, 0)),
            pl.BlockSpec((1, D), lambda i: (0, 0)),
        ],
        out_specs=[
            pl.BlockSpec((RB_H, D), lambda i: (i, 0)),
            pl.BlockSpec((RB_H, 1), lambda i: (i, 0)),
            pl.BlockSpec((RB_H, 1), lambda i: (i, 0)),
        ],
        out_shape=[
            jax.ShapeDtypeStruct((N, D), jnp.bfloat16),
            jax.ShapeDtypeStruct((N, 1), jnp.float32),
            jax.ShapeDtypeStruct((N, 1), jnp.float32),
        ],
    )(x, W, attn_src, attn_dst)

    a_src_row = a_src.reshape(1, N)

    out = pl.pallas_call(
        _attn_kernel,
        grid=(N // RB,),
        in_specs=[
            pl.BlockSpec((RB, N), lambda i: (i, 0)),
            pl.BlockSpec((1, N), lambda i: (0, 0)),
            pl.BlockSpec((RB, 1), lambda i: (i, 0)),
            pl.BlockSpec((N, D), lambda i: (0, 0)),
        ],
        out_specs=pl.BlockSpec((RB, D), lambda i: (i, 0)),
        out_shape=jax.ShapeDtypeStruct((N, D), jnp.float32),
    )(adj, a_src_row, a_dst, hbf)
    return out
